# Initial kernel scaffold; baseline (speedup 1.0000x reference)
import jax
import jax.numpy as jnp
from jax.experimental import pallas as pl


def kernel(x, edge_index, W1, b1, W2, b2, W3, b3):
    raise NotImplementedError("write your pallas kernel here")


# speedup vs baseline: 13.0321x; 13.0321x over previous
import functools

import jax
import jax.numpy as jnp
from jax import lax
from jax.experimental import pallas as pl
from jax.experimental.pallas import tpu as pltpu
from jax.experimental.pallas import tpu_sc as plsc

N = 10000
E = 320000
D = 128
NC = 2
NS = 16
CH = 128
NCH = (E + NC * NS * CH - 1) // (NC * NS * CH)
EPAD = NC * NS * NCH * CH
NP = 10240
RPT = NP // NS

_mesh = plsc.VectorSubcoreMesh(core_axis_name="c", subcore_axis_name="s")


@functools.partial(
    pl.kernel,
    out_type=jax.ShapeDtypeStruct((NC, NP, 16), jnp.float32),
    mesh=_mesh,
    scratch_types=[
        pltpu.VMEM((NCH, CH), jnp.int32),
        pltpu.VMEM((CH, 16), jnp.float32),
        pltpu.VMEM((16, 16), jnp.float32),
        pltpu.VMEM_SHARED((NP, 16), jnp.float32),
    ],
)
def _deg_kernel(dst_hbm, out_hbm, dst_v, ones_v, zb_v, acc):
    c = lax.axis_index("c")
    s = lax.axis_index("s")
    for r in range(16):
        zb_v[r, :] = jnp.zeros((16,), jnp.float32)
    for r in range(CH):
        ones_v[r, :] = jnp.ones((16,), jnp.float32)

    def _zero(i, _):
        pltpu.sync_copy(zb_v, acc.at[pl.ds(s * RPT + i * 16, 16)])
        return 0
    lax.fori_loop(0, RPT // 16, _zero, 0)
    plsc.subcore_barrier()

    pltpu.sync_copy(dst_hbm.at[c, s], dst_v)

    def _body(j, _):
        pltpu.sync_copy(ones_v, acc.at[dst_v.at[j]], add=True)
        return 0
    lax.fori_loop(0, NCH, _body, 0)
    plsc.subcore_barrier()

    pltpu.sync_copy(acc.at[pl.ds(s * RPT, RPT)], out_hbm.at[c, pl.ds(s * RPT, RPT)])


@functools.partial(
    pl.kernel,
    out_type=jax.ShapeDtypeStruct((NC, NP, D), jnp.float32),
    mesh=_mesh,
    scratch_types=[
        pltpu.VMEM((NCH, CH), jnp.int32),
        pltpu.VMEM((NCH, CH), jnp.int32),
        pltpu.VMEM((CH, D), jnp.float32),
        pltpu.VMEM((16, D), jnp.float32),
        pltpu.VMEM_SHARED((NP, D), jnp.float32),
        pltpu.SemaphoreType.DMA,
    ],
)
def _agg_kernel(src_hbm, dst_hbm, g_hbm, out_hbm, src_v, dst_v, rows_v, zb_v, acc, sem):
    c = lax.axis_index("c")
    s = lax.axis_index("s")
    for r in range(16):
        for q in range(D // 16):
            zb_v[r, pl.ds(q * 16, 16)] = jnp.zeros((16,), jnp.float32)

    def _zero(i, _):
        pltpu.sync_copy(zb_v, acc.at[pl.ds(s * RPT + i * 16, 16)])
        return 0
    lax.fori_loop(0, RPT // 16, _zero, 0)
    plsc.subcore_barrier()

    pltpu.sync_copy(src_hbm.at[c, s], src_v)
    pltpu.sync_copy(dst_hbm.at[c, s], dst_v)

    def _body(j, _):
        pltpu.async_copy(g_hbm.at[src_v.at[j]], rows_v, sem).wait()
        pltpu.sync_copy(rows_v, acc.at[dst_v.at[j]], add=True)
        return 0
    lax.fori_loop(0, NCH, _body, 0)
    plsc.subcore_barrier()

    pltpu.sync_copy(acc.at[pl.ds(s * RPT, RPT)], out_hbm.at[c, pl.ds(s * RPT, RPT)])


_RB = 2000
_GRID = N // _RB

_deg_spec = pl.BlockSpec((NC, _RB, 16), lambda i: (0, i, 0))
_row_spec = pl.BlockSpec((_RB, D), lambda i: (i, 0))
_par_spec = pl.BlockSpec((NC, _RB, D), lambda i: (0, i, 0))
_mat_spec = pl.BlockSpec((D, D), lambda i: (0, 0))


def _dinv_of(degp_ref):
    deg = degp_ref[0, :, 0:1] + degp_ref[1, :, 0:1] + 1.0
    return lax.rsqrt(jnp.maximum(deg, 1e-12))


def _stage1_body(x_ref, w1_ref, degp_ref, g1_ref):
    dinv = _dinv_of(degp_ref)
    g1_ref[...] = jnp.dot(x_ref[...], w1_ref[...],
                          preferred_element_type=jnp.float32) * dinv


def _stage2_body(p_ref, g1_ref, degp_ref, b1_ref, w2_ref, g2_ref):
    dinv = _dinv_of(degp_ref)
    agg = (p_ref[0] + p_ref[1] + g1_ref[...]) * dinv + b1_ref[...]
    h = jnp.maximum(agg, 0.0)
    g2_ref[...] = jnp.dot(h, w2_ref[...],
                          preferred_element_type=jnp.float32) * dinv


def _stage3_body(p_ref, g2_ref, degp_ref, b2_ref, w3_ref, b3_ref, out_ref):
    dinv = _dinv_of(degp_ref)
    agg = (p_ref[0] + p_ref[1] + g2_ref[...]) * dinv + b2_ref[...]
    h = jnp.maximum(agg, 0.0)
    out_ref[...] = jnp.dot(h, w3_ref[...],
                           preferred_element_type=jnp.float32) + b3_ref[...]


_stage1 = pl.pallas_call(
    _stage1_body,
    grid=(_GRID,),
    in_specs=[_row_spec, _mat_spec, _deg_spec],
    out_specs=_row_spec,
    out_shape=jax.ShapeDtypeStruct((N, D), jnp.float32),
)

_stage2 = pl.pallas_call(
    _stage2_body,
    grid=(_GRID,),
    in_specs=[_par_spec, _row_spec, _deg_spec,
              pl.BlockSpec((1, D), lambda i: (0, 0)), _mat_spec],
    out_specs=_row_spec,
    out_shape=jax.ShapeDtypeStruct((N, D), jnp.float32),
)

_stage3 = pl.pallas_call(
    _stage3_body,
    grid=(_GRID,),
    in_specs=[_par_spec, _row_spec, _deg_spec,
              pl.BlockSpec((1, D), lambda i: (0, 0)),
              pl.BlockSpec((D, 1), lambda i: (0, 0)),
              pl.BlockSpec((1, 1), lambda i: (0, 0))],
    out_specs=pl.BlockSpec((_RB, 1), lambda i: (i, 0)),
    out_shape=jax.ShapeDtypeStruct((N, 1), jnp.float32),
)


def kernel(x, edge_index, W1, b1, W2, b2, W3, b3):
    src = edge_index[0]
    dst = edge_index[1]
    pad = EPAD - E
    srcp = jnp.concatenate(
        [src, jnp.zeros((pad,), jnp.int32)]).reshape(NC, NS, NCH, CH)
    dstp = jnp.concatenate(
        [dst, jnp.full((pad,), N, jnp.int32)]).reshape(NC, NS, NCH, CH)

    degp = _deg_kernel(dstp)

    g1 = _stage1(x, W1, degp)
    p1 = _agg_kernel(srcp, dstp, g1)
    g2 = _stage2(p1, g1, degp, b1.reshape(1, D), W2)
    p2 = _agg_kernel(srcp, dstp, g2)
    out = _stage3(p2, g2, degp, b2.reshape(1, D), W3, b3.reshape(1, 1))
    return out
